# baseline (speedup 1.0000x reference)
import functools

import jax
import jax.numpy as jnp
from jax import lax
from jax.experimental import pallas as pl
from jax.experimental.pallas import tpu as pltpu
from jax.experimental.pallas import tpu_sc as plsc

HIDDEN = 768
BATCH = 4
SEQ = 2048
ROWS = BATCH * SEQ
EPS = 1e-12

_info = plsc.get_sparse_core_info()
_NC, _NS = _info.num_cores, _info.num_subcores
_NW = _NC * _NS
_CH = 64

_mesh = plsc.VectorSubcoreMesh(core_axis_name="c", subcore_axis_name="s")


@functools.lru_cache(maxsize=None)
def _make_gather_sc(nrows):
    rpw = nrows // _NW
    nch = max(1, rpw // _CH)
    ch = rpw // nch

    @functools.partial(
        pl.kernel,
        mesh=_mesh,
        out_type=jax.ShapeDtypeStruct((nrows, HIDDEN), jnp.float32),
        scratch_types=[
            pltpu.VMEM((rpw,), jnp.int32),
            pltpu.VMEM((ch, HIDDEN), jnp.float32),
            pltpu.VMEM((ch, HIDDEN), jnp.float32),
            pltpu.SemaphoreType.DMA,
            pltpu.SemaphoreType.DMA,
        ],
    )
    def _gather_sc(table_hbm, ids_hbm, out_hbm, idx_v, buf0, buf1, sem0, sem1):
        wid = lax.axis_index("s") * _NC + lax.axis_index("c")
        base = wid * rpw
        pltpu.sync_copy(ids_hbm.at[pl.ds(base, rpw)], idx_v)
        bufs = (buf0, buf1)
        sems = (sem0, sem1)
        cur = pltpu.async_copy(table_hbm.at[idx_v.at[pl.ds(0, ch)]], bufs[0], sems[0])
        for c in range(nch):
            nxt = None
            if c + 1 < nch:
                nxt = pltpu.async_copy(
                    table_hbm.at[idx_v.at[pl.ds((c + 1) * ch, ch)]],
                    bufs[(c + 1) % 2],
                    sems[(c + 1) % 2],
                )
            cur.wait()
            pltpu.sync_copy(bufs[c % 2], out_hbm.at[pl.ds(base + c * ch, ch)])
            cur = nxt

    return _gather_sc


def _ln_body(pos_ref, gamma_ref, beta_ref, emb_ref, out_ref):
    x = emb_ref[...] + pos_ref[...]
    mean = jnp.mean(x, axis=1, keepdims=True)
    xc = x - mean
    var = jnp.mean(xc * xc, axis=1, keepdims=True)
    inv = lax.rsqrt(var + EPS)
    out_ref[...] = xc * inv * gamma_ref[...] + beta_ref[...]


def _ln_tc(pos_table, gb, bb, emb, nrows):
    r = min(2048, nrows)
    return pl.pallas_call(
        _ln_body,
        grid=(SEQ // r, nrows // SEQ if nrows >= SEQ else 1),
        in_specs=[
            pl.BlockSpec((r, HIDDEN), lambda j, b: (j, 0)),
            pl.BlockSpec((1, HIDDEN), lambda j, b: (0, 0)),
            pl.BlockSpec((1, HIDDEN), lambda j, b: (0, 0)),
            pl.BlockSpec((r, HIDDEN), lambda j, b: (b * (SEQ // r) + j, 0)),
        ],
        out_specs=pl.BlockSpec((r, HIDDEN), lambda j, b: (b * (SEQ // r) + j, 0)),
        out_shape=jax.ShapeDtypeStruct((nrows, HIDDEN), jnp.float32),
    )(pos_table, gb, bb, emb)


def kernel(input_ids, token_table, pos_table, gamma, beta):
    ids = input_ids.reshape(-1).astype(jnp.int32)
    gb = gamma.reshape(1, HIDDEN)
    bb = beta.reshape(1, HIDDEN)
    nchunks = 2
    rows_c = ROWS // nchunks
    gather = _make_gather_sc(rows_c)
    outs = []
    for c in range(nchunks):
        ids_c = lax.slice_in_dim(ids, c * rows_c, (c + 1) * rows_c, axis=0)
        emb_c = gather(token_table, ids_c)
        outs.append(_ln_tc(pos_table, gb, bb, emb_c, rows_c))
    return jnp.concatenate(outs, axis=0).reshape(BATCH, SEQ, HIDDEN)
